# baseline (speedup 1.0000x reference)
import functools
import math

import jax
import jax.numpy as jnp
from jax import lax
from jax.experimental import pallas as pl
from jax.experimental.pallas import tpu as pltpu
from jax.experimental.pallas import tpu_sc as plsc

D_MODEL = 64
SCALE = math.sqrt(D_MODEL)

_NUM_CORES = 2
_NUM_SUBCORES = 16
_NW = _NUM_CORES * _NUM_SUBCORES

_SUB = 128
_L = 16
_WIDE = 2 * D_MODEL
_NBUF = 2


@functools.partial(jax.jit, static_argnames=("n",))
def _embed_wide(table_w, wid, poff, *, n):
    npw = n // _NW
    nsub = npw // _SUB

    mesh = plsc.VectorSubcoreMesh(core_axis_name="c", subcore_axis_name="s")

    @functools.partial(
        pl.kernel,
        out_type=jax.ShapeDtypeStruct((n * D_MODEL,), jnp.float32),
        mesh=mesh,
        compiler_params=pltpu.CompilerParams(use_tc_tiling_on_sc=False),
        scratch_types=[
            pltpu.VMEM((npw,), jnp.int32),
            pltpu.VMEM((_SUB, _WIDE), jnp.float32),
            pltpu.VMEM((_SUB, _WIDE), jnp.float32),
            pltpu.VMEM((_SUB * D_MODEL,), jnp.float32),
            pltpu.VMEM((_SUB * D_MODEL,), jnp.float32),
            pltpu.VMEM((_SUB * _L,), jnp.int32),
            pltpu.VMEM((_SUB * _L,), jnp.int32),
            pltpu.SemaphoreType.DMA,
            pltpu.SemaphoreType.DMA,
            pltpu.SemaphoreType.DMA,
            pltpu.SemaphoreType.DMA,
            pltpu.SemaphoreType.DMA,
            pltpu.SemaphoreType.DMA,
        ],
    )
    def emb(table_hbm, wid_hbm, prep_hbm, out_hbm, wid_v,
            gb0, gb1, ob0, ob1, pr0, pr1,
            gs0, gs1, ss0, ss1, ps0, ps1):
        worker = lax.axis_index("s") * _NUM_CORES + lax.axis_index("c")
        base = worker * npw
        pltpu.sync_copy(wid_hbm.at[pl.ds(base, npw)], wid_v)

        gbuf = (gb0, gb1)
        obuf = (ob0, ob1)
        pbuf = (pr0, pr1)
        gsem = (gs0, gs1)
        ssem = (ss0, ss1)
        psem = (ps0, ps1)

        def start_gather(g, b):
            pltpu.async_copy(
                table_hbm.at[wid_v.at[pl.ds(g * _SUB, _SUB)]],
                gbuf[b], gsem[b])
            pltpu.async_copy(
                prep_hbm.at[pl.ds((base + g * _SUB) * _L, _SUB * _L)],
                pbuf[b], psem[b])

        def wait_gather(b):
            pltpu.make_async_copy(
                table_hbm.at[wid_v.at[pl.ds(0, _SUB)]],
                gbuf[b], gsem[b]).wait()
            pltpu.make_async_copy(
                prep_hbm.at[pl.ds(0, _SUB * _L)], pbuf[b], psem[b]).wait()

        def compact(b):
            gb = gbuf[b]
            ob = obuf[b]
            pr = pbuf[b]

            @plsc.parallel_loop(0, _SUB, unroll=2)
            def _row(r):
                mask = pr[pl.ds(r * _L, _L)] != 0
                for j in range(D_MODEL // _L):
                    lo = gb[r, pl.ds(j * _L, _L)]
                    hi = gb[r, pl.ds(D_MODEL + j * _L, _L)]
                    v = jnp.where(mask, hi, lo) * SCALE
                    ob[pl.ds(r * D_MODEL + j * _L, _L)] = v

        def start_scatter(g, b):
            pltpu.async_copy(
                obuf[b],
                out_hbm.at[pl.ds((base + g * _SUB) * D_MODEL, _SUB * D_MODEL)],
                ssem[b])

        def wait_scatter(b):
            pltpu.make_async_copy(
                obuf[b], out_hbm.at[pl.ds(0, _SUB * D_MODEL)], ssem[b]).wait()

        for b in range(_NBUF):
            start_gather(b, b)

        for g in range(_NBUF):
            b = g
            wait_gather(b)
            compact(b)
            start_scatter(g, b)
            start_gather(g + _NBUF, b)

        @pl.loop(_NBUF, nsub - _NBUF, step=_NBUF)
        def _main(gg):
            for b in range(_NBUF):
                g = gg + b
                wait_gather(b)
                wait_scatter(b)
                compact(b)
                start_scatter(g, b)
                start_gather(g + _NBUF, b)

        for k in range(_NBUF):
            g = nsub - _NBUF + k
            b = g % _NBUF
            wait_gather(b)
            wait_scatter(b)
            compact(b)
            start_scatter(g, b)

        for b in range(_NBUF):
            wait_scatter(b)

    return emb(table_w, wid, poff)


def kernel(x, table):
    n = x.size
    idx = x.reshape(n).astype(jnp.int32)
    wid = idx >> 1
    prep = jnp.broadcast_to((idx & 1)[:, None], (n, _L)).reshape(n * _L)
    table_w = table.reshape(table.shape[0] // 2, _WIDE)
    out_flat = _embed_wide(table_w, wid, prep, n=n)
    return out_flat.reshape(x.shape + (D_MODEL,))


# speedup vs baseline: 1.1032x; 1.1032x over previous
import functools
import math

import jax
import jax.numpy as jnp
from jax import lax
from jax.experimental import pallas as pl
from jax.experimental.pallas import tpu as pltpu
from jax.experimental.pallas import tpu_sc as plsc

D_MODEL = 64
SCALE = math.sqrt(D_MODEL)

_NUM_CORES = 2
_NUM_SUBCORES = 16
_NW = _NUM_CORES * _NUM_SUBCORES

_SUB = 128
_L = 16
_NBUF = 2


@functools.partial(jax.jit, static_argnames=("n",))
def _embed(table, idx, *, n):
    npw = n // _NW
    nsub = npw // _SUB

    mesh = plsc.VectorSubcoreMesh(core_axis_name="c", subcore_axis_name="s")

    @functools.partial(
        pl.kernel,
        out_type=jax.ShapeDtypeStruct((n, D_MODEL), jnp.float32),
        mesh=mesh,
        compiler_params=pltpu.CompilerParams(use_tc_tiling_on_sc=False),
        scratch_types=[
            pltpu.VMEM((npw,), jnp.int32),
            pltpu.VMEM((_SUB, D_MODEL), jnp.float32),
            pltpu.VMEM((_SUB, D_MODEL), jnp.float32),
            pltpu.VMEM((_SUB, D_MODEL), jnp.float32),
            pltpu.VMEM((_SUB, D_MODEL), jnp.float32),
            pltpu.SemaphoreType.DMA,
            pltpu.SemaphoreType.DMA,
            pltpu.SemaphoreType.DMA,
            pltpu.SemaphoreType.DMA,
        ],
    )
    def emb(table_hbm, idx_hbm, out_hbm, idx_v,
            gb0, gb1, ob0, ob1, gs0, gs1, ss0, ss1):
        worker = lax.axis_index("s") * _NUM_CORES + lax.axis_index("c")
        base = worker * npw
        pltpu.sync_copy(idx_hbm.at[pl.ds(base, npw)], idx_v)

        gbuf = (gb0, gb1)
        obuf = (ob0, ob1)
        gsem = (gs0, gs1)
        ssem = (ss0, ss1)

        def start_gather(g, b):
            pltpu.async_copy(
                table_hbm.at[idx_v.at[pl.ds(g * _SUB, _SUB)]],
                gbuf[b], gsem[b])

        def wait_gather(b):
            pltpu.make_async_copy(
                table_hbm.at[idx_v.at[pl.ds(0, _SUB)]],
                gbuf[b], gsem[b]).wait()

        def scale(b):
            gb = gbuf[b]
            ob = obuf[b]

            @plsc.parallel_loop(0, _SUB, unroll=2)
            def _row(r):
                for j in range(D_MODEL // _L):
                    ob[r, pl.ds(j * _L, _L)] = gb[r, pl.ds(j * _L, _L)] * SCALE

        def start_scatter(g, b):
            pltpu.async_copy(
                obuf[b],
                out_hbm.at[pl.ds(base + g * _SUB, _SUB)],
                ssem[b])

        def wait_scatter(b):
            pltpu.make_async_copy(
                obuf[b], out_hbm.at[pl.ds(0, _SUB)], ssem[b]).wait()

        for b in range(_NBUF):
            start_gather(b, b)

        for g in range(_NBUF):
            b = g
            wait_gather(b)
            scale(b)
            start_scatter(g, b)
            start_gather(g + _NBUF, b)

        @pl.loop(_NBUF, nsub - _NBUF, step=_NBUF)
        def _main(gg):
            for b in range(_NBUF):
                g = gg + b
                wait_gather(b)
                wait_scatter(b)
                scale(b)
                start_scatter(g, b)
                start_gather(g + _NBUF, b)

        for k in range(_NBUF):
            g = nsub - _NBUF + k
            b = g % _NBUF
            wait_gather(b)
            wait_scatter(b)
            scale(b)
            start_scatter(g, b)

        for b in range(_NBUF):
            wait_scatter(b)

    return emb(table, idx)


def kernel(x, table):
    n = x.size
    idx = x.reshape(n).astype(jnp.int32)
    out = _embed(table, idx, n=n)
    return out.reshape(x.shape + (D_MODEL,))
